# baseline (speedup 1.0000x reference)
import functools

import jax
import jax.numpy as jnp
from jax import lax
from jax.experimental import pallas as pl
from jax.experimental.pallas import tpu as pltpu
from jax.experimental.pallas import tpu_sc as plsc

NC = 2
NS = 16
CHUNK = 128


LOAD = 128


def _sc_segment_partials(occ, idx, nr_pad):
    n, d = occ.shape
    assert n % LOAD == 0
    nloads_total = n // LOAD
    nw = NC * NS
    max_loads = (nloads_total + nw - 1) // nw
    rows_per_tile = nr_pad // NS
    assert rows_per_tile * NS == nr_pad and rows_per_tile % 8 == 0
    zrows = 128
    assert rows_per_tile % zrows == 0
    zeros = jnp.zeros((zrows, d), jnp.float32)

    mesh = plsc.VectorSubcoreMesh(core_axis_name="c", subcore_axis_name="s")

    @functools.partial(
        pl.kernel,
        out_type=jax.ShapeDtypeStruct((NC, nr_pad, d), jnp.float32),
        mesh=mesh,
        scratch_types=[
            pltpu.VMEM_SHARED((nr_pad, d), jnp.float32),
            pltpu.VMEM((2, LOAD, d), jnp.float32),
            pltpu.VMEM((2, CHUNK), jnp.int32),
            pltpu.SemaphoreType.DMA((2,)),
            pltpu.SemaphoreType.DMA((2,)),
            pltpu.SemaphoreType.DMA,
        ],
    )
    def k(occ_hbm, idx_hbm, zeros_hbm, part_hbm, acc, rows_v, idx_v,
          rsem, isem, zsem):
        cid = lax.axis_index("c")
        sid = lax.axis_index("s")
        wid = sid * NC + cid
        nl = (nloads_total - wid + nw - 1) // nw

        def start_load(slot, li):
            c = wid + li * nw
            pltpu.async_copy(occ_hbm.at[pl.ds(c * LOAD, LOAD)],
                             rows_v.at[slot], rsem.at[slot])
            pltpu.async_copy(idx_hbm.at[pl.ds(c * CHUNK, CHUNK)],
                             idx_v.at[slot], isem.at[slot])

        def wait_load(slot):
            pltpu.make_async_copy(occ_hbm.at[pl.ds(0, LOAD)],
                                  rows_v.at[slot], rsem.at[slot]).wait()
            pltpu.make_async_copy(idx_hbm.at[pl.ds(0, CHUNK)],
                                  idx_v.at[slot], isem.at[slot]).wait()

        start_load(0, 0)
        start_load(1, 1)
        for j in range(rows_per_tile // zrows):
            pltpu.async_copy(
                zeros_hbm, acc.at[pl.ds(sid * rows_per_tile + j * zrows, zrows)],
                zsem)
        for j in range(rows_per_tile // zrows):
            pltpu.make_async_copy(
                zeros_hbm, acc.at[pl.ds(sid * rows_per_tile, zrows)],
                zsem).wait()
        plsc.subcore_barrier()

        def outer(o, carry):
            for b in range(2):
                li = o * 2 + b

                @pl.when(li < nl)
                def _():
                    wait_load(b)
                    pltpu.sync_copy(rows_v.at[b], acc.at[idx_v.at[b]],
                                    add=True)

                    @pl.when(li + 2 < nl)
                    def _():
                        start_load(b, li + 2)
            return carry

        lax.fori_loop(0, (max_loads + 1) // 2, outer, 0)
        plsc.subcore_barrier()
        for j in range(rows_per_tile // zrows):
            sl = pl.ds(sid * rows_per_tile + j * zrows, zrows)
            pltpu.async_copy(acc.at[sl], part_hbm.at[cid, sl], zsem)
        for j in range(rows_per_tile // zrows):
            pltpu.make_async_copy(acc.at[pl.ds(sid * rows_per_tile, zrows)],
                                  part_hbm.at[cid, pl.ds(0, zrows)], zsem).wait()

    return k(occ, idx, zeros)


def _gate_kernel(p_ref, prev_ref, w0_ref, w1_ref, b_ref, out_ref):
    p0 = p_ref[0]
    p1 = p_ref[1]
    upd = p0 + p1
    prev = prev_ref[...]
    logits = (
        jnp.dot(prev, w0_ref[...], preferred_element_type=jnp.float32)
        + jnp.dot(upd, w1_ref[...], preferred_element_type=jnp.float32)
        + b_ref[...]
    )
    z = jax.nn.sigmoid(logits)
    out_ref[...] = z * prev + (1.0 - z) * upd


def _tc_gate(partials, prev, w_gate, b_gate):
    nr_nodes, d = prev.shape
    blk = 2000
    assert nr_nodes % blk == 0
    grid = nr_nodes // blk
    w0 = w_gate[:d]
    w1 = w_gate[d:]
    b2 = b_gate.reshape(1, d)
    return pl.pallas_call(
        _gate_kernel,
        grid=(grid,),
        in_specs=[
            pl.BlockSpec((2, blk, d), lambda i: (0, i, 0)),
            pl.BlockSpec((blk, d), lambda i: (i, 0)),
            pl.BlockSpec((d, d), lambda i: (0, 0)),
            pl.BlockSpec((d, d), lambda i: (0, 0)),
            pl.BlockSpec((1, d), lambda i: (0, 0)),
        ],
        out_specs=pl.BlockSpec((blk, d), lambda i: (i, 0)),
        out_shape=jax.ShapeDtypeStruct((nr_nodes, d), jnp.float32),
    )(partials, prev, w0, w1, b2)


def kernel(flattened_nodes_occurrences, flattened_nodes_indices,
           previous_cfg_nodes_encodings, nr_cfg_nodes, W_gate, b_gate):
    idx = flattened_nodes_indices.astype(jnp.int32)
    nr_nodes = previous_cfg_nodes_encodings.shape[0]
    nr_pad = ((nr_nodes + NS * 128 - 1) // (NS * 128)) * NS * 128
    partials = _sc_segment_partials(flattened_nodes_occurrences, idx, nr_pad)
    return _tc_gate(partials, previous_cfg_nodes_encodings, W_gate, b_gate)
